# baseline (speedup 1.0000x reference)
import functools

import jax
import jax.numpy as jnp
from jax import lax
from jax.experimental import pallas as pl
from jax.experimental.pallas import tpu as pltpu
from jax.experimental.pallas import tpu_sc as plsc

N = 320000
D = 128
S = 512

NC = 2
NS = 16
DC = D // NC
ROWS_PER_SUB = N // NS
CHUNK = 400
GROUP = 16
N_CHUNKS = ROWS_PER_SUB // CHUNK
N_GROUPS = CHUNK // GROUP
ROWS_PER_OUT = S // NS
SCAT = 128
N_SCAT = S // SCAT
NV = DC // 16


@functools.partial(
    pl.kernel,
    out_type=jax.ShapeDtypeStruct((NC, S, DC), jnp.float32),
    mesh=plsc.VectorSubcoreMesh(core_axis_name="c", subcore_axis_name="s"),
    scratch_types=[
        pltpu.VMEM((CHUNK, DC), jnp.float32),
        pltpu.VMEM((CHUNK, DC), jnp.float32),
        pltpu.VMEM((1, CHUNK), jnp.int32),
        pltpu.VMEM((1, CHUNK), jnp.int32),
        pltpu.VMEM((ROWS_PER_OUT, DC), jnp.float32),
        pltpu.VMEM((S, DC), jnp.float32),
        pltpu.VMEM((N_SCAT, SCAT), jnp.int32),
        pltpu.VMEM_SHARED((S, DC), jnp.float32),
        pltpu.SemaphoreType.DMA,
        pltpu.SemaphoreType.DMA,
        pltpu.SemaphoreType.DMA,
    ],
    compiler_params=pltpu.CompilerParams(use_tc_tiling_on_sc=False),
)
def _seg_sum(x_hbm, ids_hbm, out_hbm, buf_a, buf_b, idb_a, idb_b, obuf,
             lacc, sidx, acc, lsem_a, lsem_b, ssem):
    c = lax.axis_index("c")
    s = lax.axis_index("s")
    col0 = c * DC
    row0 = s * ROWS_PER_SUB
    idrow0 = s * N_CHUNKS

    bufs = (buf_a, buf_b)
    idbs = (idb_a, idb_b)
    lsems = (lsem_a, lsem_b)
    zvec = jnp.zeros((16,), jnp.float32)

    for r in range(ROWS_PER_OUT):
        for k in range(NV):
            obuf[r, pl.ds(k * 16, 16)] = zvec
    pltpu.sync_copy(obuf, acc.at[pl.ds(s * ROWS_PER_OUT, ROWS_PER_OUT)])

    def zero_body(r, carry):
        for k in range(NV):
            lacc[r, pl.ds(k * 16, 16)] = zvec
        return carry

    lax.fori_loop(0, S, zero_body, 0)

    for r in range(N_SCAT):
        for k in range(SCAT // 16):
            sidx[r, pl.ds(k * 16, 16)] = (
                lax.iota(jnp.int32, 16) + (r * SCAT + k * 16)
            )

    def load(j, p):
        pltpu.async_copy(
            x_hbm.at[pl.ds(row0 + j * CHUNK, CHUNK), pl.ds(col0, DC)],
            bufs[p], lsems[p],
        )
        pltpu.async_copy(
            ids_hbm.at[pl.ds(idrow0 + j, 1)], idbs[p], lsems[p],
        )

    def wait_load(p):
        pltpu.make_async_copy(
            x_hbm.at[pl.ds(row0, CHUNK), pl.ds(col0, DC)], bufs[p], lsems[p]
        ).wait()
        pltpu.make_async_copy(
            ids_hbm.at[pl.ds(idrow0, 1)], idbs[p], lsems[p]
        ).wait()

    def group_sum(buf, base, k):
        cs = pl.ds(k * 16, 16)
        t0 = [buf[base + i, cs] + buf[base + i + 8, cs] for i in range(8)]
        t1 = [t0[i] + t0[i + 4] for i in range(4)]
        t2 = [t1[0] + t1[2], t1[1] + t1[3]]
        return t2[0] + t2[1]

    def process_chunk(buf, idb):
        def group_body(g, carry):
            base = g * GROUP
            id_first = idb[0, pl.ds(base, GROUP)][0]
            for k in range(NV):
                plsc.addupdate(
                    lacc.at[id_first, pl.ds(k * 16, 16)],
                    group_sum(buf, base, k),
                )
            return carry

        lax.fori_loop(0, N_GROUPS, group_body, 0)

        def fix_body(g, carry):
            base = g * GROUP
            gv = idb[0, pl.ds(base, GROUP)]
            id_first = gv[0]

            def fix():
                for k in range(NV):
                    plsc.addupdate(
                        lacc.at[id_first, pl.ds(k * 16, 16)],
                        -group_sum(buf, base, k),
                    )
                for i in range(GROUP):
                    rid = gv[i]
                    for k in range(NV):
                        cs = pl.ds(k * 16, 16)
                        plsc.addupdate(lacc.at[rid, cs], buf[base + i, cs])

            pl.when(id_first != gv[GROUP - 1])(fix)
            return carry

        lax.fori_loop(0, N_GROUPS, fix_body, 0)

    load(0, 0)

    def pair_body(g, carry):
        j0 = 2 * g
        load(j0 + 1, 1)
        wait_load(0)
        process_chunk(bufs[0], idbs[0])
        load(jnp.minimum(j0 + 2, N_CHUNKS - 1), 0)
        wait_load(1)
        process_chunk(bufs[1], idbs[1])
        return carry

    lax.fori_loop(0, N_CHUNKS // 2, pair_body, 0)
    wait_load(0)

    scatd = [
        pltpu.async_copy(
            lacc.at[pl.ds(r * SCAT, SCAT)], acc.at[sidx.at[r]], ssem,
            add=True,
        )
        for r in range(N_SCAT)
    ]
    for d in scatd:
        d.wait()

    plsc.subcore_barrier()

    pltpu.sync_copy(acc.at[pl.ds(s * ROWS_PER_OUT, ROWS_PER_OUT)], obuf)
    pltpu.sync_copy(
        obuf, out_hbm.at[c, pl.ds(s * ROWS_PER_OUT, ROWS_PER_OUT)]
    )


def kernel(x, batch):
    ids = batch.astype(jnp.int32).reshape(N // CHUNK, CHUNK)
    halves = _seg_sum(x, ids)
    return jnp.concatenate([halves[0], halves[1]], axis=1)


# speedup vs baseline: 1.0702x; 1.0702x over previous
import functools

import jax
import jax.numpy as jnp
from jax import lax
from jax.experimental import pallas as pl
from jax.experimental.pallas import tpu as pltpu
from jax.experimental.pallas import tpu_sc as plsc

N = 320000
D = 128
S = 512

NC = 2
NS = 16
DC = D // NC
ROWS_PER_SUB = N // NS
CHUNK = 400
GROUP = 16
N_CHUNKS = ROWS_PER_SUB // CHUNK
N_GROUPS = CHUNK // GROUP
SUPER = 5
N_SUPER = N_CHUNKS // SUPER
SUB = 100
N_SUB = CHUNK // SUB
ROWS_PER_OUT = S // NS
SCAT = 128
N_SCAT = S // SCAT
NV = DC // 16


@functools.partial(
    pl.kernel,
    out_type=jax.ShapeDtypeStruct((NC, S, DC), jnp.float32),
    mesh=plsc.VectorSubcoreMesh(core_axis_name="c", subcore_axis_name="s"),
    scratch_types=[
        pltpu.VMEM((CHUNK, DC), jnp.float32),
        pltpu.VMEM((CHUNK, DC), jnp.float32),
        pltpu.VMEM((CHUNK, DC), jnp.float32),
        pltpu.VMEM((1, CHUNK), jnp.int32),
        pltpu.VMEM((1, CHUNK), jnp.int32),
        pltpu.VMEM((N_SUB, SUB), jnp.int32),
        pltpu.VMEM((ROWS_PER_OUT, DC), jnp.float32),
        pltpu.VMEM((S, DC), jnp.float32),
        pltpu.VMEM((N_SCAT, SCAT), jnp.int32),
        pltpu.VMEM_SHARED((S, DC), jnp.float32),
        pltpu.SemaphoreType.DMA,
        pltpu.SemaphoreType.DMA,
        pltpu.SemaphoreType.DMA,
        pltpu.SemaphoreType.DMA,
        pltpu.SemaphoreType.DMA,
    ],
    compiler_params=pltpu.CompilerParams(use_tc_tiling_on_sc=False),
)
def _seg_sum(x_hbm, ids_hbm, ids2_hbm, out_hbm, buf_a, buf_b, buf_c,
             idb_a, idb_b, idb_c, obuf, lacc, sidx, acc,
             lsem_a, lsem_b, lsem_c, csem, ssem):
    c = lax.axis_index("c")
    s = lax.axis_index("s")
    col0 = c * DC
    row0 = s * ROWS_PER_SUB
    idrow0 = s * N_CHUNKS

    bufs = (buf_a, buf_b)
    idbs = (idb_a, idb_b)
    lsems = (lsem_a, lsem_b)
    zvec = jnp.zeros((16,), jnp.float32)

    for r in range(ROWS_PER_OUT):
        for k in range(NV):
            obuf[r, pl.ds(k * 16, 16)] = zvec
    pltpu.sync_copy(obuf, acc.at[pl.ds(s * ROWS_PER_OUT, ROWS_PER_OUT)])

    def zero_body(r, carry):
        for k in range(NV):
            lacc[r, pl.ds(k * 16, 16)] = zvec
        return carry

    lax.fori_loop(0, S, zero_body, 0)

    for r in range(N_SCAT):
        for k in range(SCAT // 16):
            sidx[r, pl.ds(k * 16, 16)] = (
                lax.iota(jnp.int32, 16) + (r * SCAT + k * 16)
            )

    plsc.subcore_barrier()

    def load(j, p):
        pltpu.async_copy(
            x_hbm.at[pl.ds(row0 + j * CHUNK, CHUNK), pl.ds(col0, DC)],
            bufs[p], lsems[p],
        )
        pltpu.async_copy(
            ids_hbm.at[pl.ds(idrow0 + j, 1)], idbs[p], lsems[p],
        )

    def wait_load(p):
        pltpu.make_async_copy(
            x_hbm.at[pl.ds(row0, CHUNK), pl.ds(col0, DC)], bufs[p], lsems[p]
        ).wait()
        pltpu.make_async_copy(
            ids_hbm.at[pl.ds(idrow0, 1)], idbs[p], lsems[p]
        ).wait()

    def load_c(j):
        pltpu.async_copy(
            x_hbm.at[pl.ds(row0 + j * CHUNK, CHUNK), pl.ds(col0, DC)],
            buf_c, lsem_c,
        )
        pltpu.async_copy(
            ids2_hbm.at[pl.ds((idrow0 + j) * N_SUB, N_SUB)], idb_c, lsem_c,
        )

    def wait_load_c():
        pltpu.make_async_copy(
            x_hbm.at[pl.ds(row0, CHUNK), pl.ds(col0, DC)], buf_c, lsem_c
        ).wait()
        pltpu.make_async_copy(
            ids2_hbm.at[pl.ds(idrow0, N_SUB)], idb_c, lsem_c
        ).wait()

    def group_sum(buf, base, k):
        cs = pl.ds(k * 16, 16)
        t0 = [buf[base + i, cs] + buf[base + i + 8, cs] for i in range(8)]
        t1 = [t0[i] + t0[i + 4] for i in range(4)]
        t2 = [t1[0] + t1[2], t1[1] + t1[3]]
        return t2[0] + t2[1]

    def process_chunk(buf, idb):
        def group_body(g, carry):
            base = g * GROUP
            id_first = idb[0, pl.ds(base, GROUP)][0]
            for k in range(NV):
                plsc.addupdate(
                    lacc.at[id_first, pl.ds(k * 16, 16)],
                    group_sum(buf, base, k),
                )
            return carry

        lax.fori_loop(0, N_GROUPS, group_body, 0)

        def fix_body(g, carry):
            base = g * GROUP
            gv = idb[0, pl.ds(base, GROUP)]
            id_first = gv[0]

            def fix():
                for k in range(NV):
                    plsc.addupdate(
                        lacc.at[id_first, pl.ds(k * 16, 16)],
                        -group_sum(buf, base, k),
                    )
                for i in range(GROUP):
                    rid = gv[i]
                    for k in range(NV):
                        cs = pl.ds(k * 16, 16)
                        plsc.addupdate(lacc.at[rid, cs], buf[base + i, cs])

            pl.when(id_first != gv[GROUP - 1])(fix)
            return carry

        lax.fori_loop(0, N_GROUPS, fix_body, 0)

    load_c(0)
    load(1, 0)

    def super_body(g, carry):
        j0 = g * SUPER
        wait_load_c()
        for k in range(N_SUB):
            pltpu.async_copy(
                buf_c.at[pl.ds(k * SUB, SUB)], acc.at[idb_c.at[k]],
                csem, add=True,
            )
        load(j0 + 2, 1)
        wait_load(0)
        process_chunk(bufs[0], idbs[0])
        load(j0 + 3, 0)
        wait_load(1)
        process_chunk(bufs[1], idbs[1])
        load(j0 + 4, 1)
        wait_load(0)
        process_chunk(bufs[0], idbs[0])
        for k in range(N_SUB):
            pltpu.make_async_copy(
                buf_c.at[pl.ds(0, SUB)], acc.at[idb_c.at[0]], csem
            ).wait()
        load_c(jnp.minimum(j0 + SUPER, N_CHUNKS - 1))
        load(jnp.minimum(j0 + SUPER + 1, N_CHUNKS - 1), 0)
        wait_load(1)
        process_chunk(bufs[1], idbs[1])
        return carry

    lax.fori_loop(0, N_SUPER, super_body, 0)
    wait_load_c()
    wait_load(0)

    scatd = [
        pltpu.async_copy(
            lacc.at[pl.ds(r * SCAT, SCAT)], acc.at[sidx.at[r]], ssem,
            add=True,
        )
        for r in range(N_SCAT)
    ]
    for d in scatd:
        d.wait()

    plsc.subcore_barrier()

    pltpu.sync_copy(acc.at[pl.ds(s * ROWS_PER_OUT, ROWS_PER_OUT)], obuf)
    pltpu.sync_copy(
        obuf, out_hbm.at[c, pl.ds(s * ROWS_PER_OUT, ROWS_PER_OUT)]
    )


def kernel(x, batch):
    ids = batch.astype(jnp.int32)
    halves = _seg_sum(
        x, ids.reshape(N // CHUNK, CHUNK), ids.reshape(N // SUB, SUB)
    )
    return jnp.concatenate([halves[0], halves[1]], axis=1)


# speedup vs baseline: 1.1147x; 1.0416x over previous
import functools

import jax
import jax.numpy as jnp
from jax import lax
from jax.experimental import pallas as pl
from jax.experimental.pallas import tpu as pltpu
from jax.experimental.pallas import tpu_sc as plsc

N = 320000
D = 128
S = 512

NC = 2
NS = 16
DC = D // NC
ROWS_PER_SUB = N // NS
CHUNK = 400
GROUP = 16
N_CHUNKS = ROWS_PER_SUB // CHUNK
N_GROUPS = CHUNK // GROUP
SUPER = 5
N_SUPER = N_CHUNKS // SUPER
SUB = 100
N_SUB = CHUNK // SUB
ROWS_PER_OUT = S // NS
SCAT = 128
N_SCAT = S // SCAT
NV = DC // 16


@functools.partial(
    pl.kernel,
    out_type=jax.ShapeDtypeStruct((NC, S, DC), jnp.float32),
    mesh=plsc.VectorSubcoreMesh(core_axis_name="c", subcore_axis_name="s"),
    scratch_types=[
        pltpu.VMEM((CHUNK, DC), jnp.float32),
        pltpu.VMEM((CHUNK, DC), jnp.float32),
        pltpu.VMEM((CHUNK, DC), jnp.float32),
        pltpu.VMEM((1, CHUNK), jnp.int32),
        pltpu.VMEM((1, CHUNK), jnp.int32),
        pltpu.VMEM((N_SUB, SUB), jnp.int32),
        pltpu.VMEM((ROWS_PER_OUT, DC), jnp.float32),
        pltpu.VMEM((S, DC), jnp.float32),
        pltpu.VMEM((N_SCAT, SCAT), jnp.int32),
        pltpu.VMEM_SHARED((S, DC), jnp.float32),
        pltpu.SemaphoreType.DMA,
        pltpu.SemaphoreType.DMA,
        pltpu.SemaphoreType.DMA,
        pltpu.SemaphoreType.DMA,
        pltpu.SemaphoreType.DMA,
    ],
    compiler_params=pltpu.CompilerParams(use_tc_tiling_on_sc=False),
)
def _seg_sum(x_hbm, ids_hbm, ids2_hbm, out_hbm, buf_a, buf_b, buf_c,
             idb_a, idb_b, idb_c, obuf, lacc, sidx, acc,
             lsem_a, lsem_b, lsem_c, csem, ssem):
    c = lax.axis_index("c")
    s = lax.axis_index("s")
    col0 = c * DC
    row0 = s * ROWS_PER_SUB
    idrow0 = s * N_CHUNKS

    bufs = (buf_a, buf_b)
    idbs = (idb_a, idb_b)
    lsems = (lsem_a, lsem_b)
    zvec = jnp.zeros((16,), jnp.float32)

    for r in range(ROWS_PER_OUT):
        for k in range(NV):
            obuf[r, pl.ds(k * 16, 16)] = zvec
    pltpu.sync_copy(obuf, acc.at[pl.ds(s * ROWS_PER_OUT, ROWS_PER_OUT)])

    def zero_body(r, carry):
        for k in range(NV):
            lacc[r, pl.ds(k * 16, 16)] = zvec
        return carry

    lax.fori_loop(0, S, zero_body, 0)

    for r in range(N_SCAT):
        for k in range(SCAT // 16):
            sidx[r, pl.ds(k * 16, 16)] = (
                lax.iota(jnp.int32, 16) + (r * SCAT + k * 16)
            )

    plsc.subcore_barrier()

    def load(j, p):
        pltpu.async_copy(
            x_hbm.at[pl.ds(row0 + j * CHUNK, CHUNK), pl.ds(col0, DC)],
            bufs[p], lsems[p],
        )
        pltpu.async_copy(
            ids_hbm.at[pl.ds(idrow0 + j, 1)], idbs[p], lsems[p],
        )

    def wait_load(p):
        pltpu.make_async_copy(
            x_hbm.at[pl.ds(row0, CHUNK), pl.ds(col0, DC)], bufs[p], lsems[p]
        ).wait()
        pltpu.make_async_copy(
            ids_hbm.at[pl.ds(idrow0, 1)], idbs[p], lsems[p]
        ).wait()

    def load_c(j):
        pltpu.async_copy(
            x_hbm.at[pl.ds(row0 + j * CHUNK, CHUNK), pl.ds(col0, DC)],
            buf_c, lsem_c,
        )
        pltpu.async_copy(
            ids2_hbm.at[pl.ds((idrow0 + j) * N_SUB, N_SUB)], idb_c, lsem_c,
        )

    def wait_load_c():
        pltpu.make_async_copy(
            x_hbm.at[pl.ds(row0, CHUNK), pl.ds(col0, DC)], buf_c, lsem_c
        ).wait()
        pltpu.make_async_copy(
            ids2_hbm.at[pl.ds(idrow0, N_SUB)], idb_c, lsem_c
        ).wait()

    def group_sum(buf, base, k):
        cs = pl.ds(k * 16, 16)
        t0 = [buf[base + i, cs] + buf[base + i + 8, cs] for i in range(8)]
        t1 = [t0[i] + t0[i + 4] for i in range(4)]
        t2 = [t1[0] + t1[2], t1[1] + t1[3]]
        return t2[0] + t2[1]

    def process_chunk(buf, idb):
        def group_body(g, carry):
            base = g * GROUP
            gv = idb[0, pl.ds(base, GROUP)]
            id_first = gv[0]
            uniform = id_first == gv[GROUP - 1]

            def fast():
                for k in range(NV):
                    plsc.addupdate(
                        lacc.at[id_first, pl.ds(k * 16, 16)],
                        group_sum(buf, base, k),
                    )

            def slow():
                for i in range(GROUP):
                    rid = gv[i]
                    for k in range(NV):
                        cs = pl.ds(k * 16, 16)
                        plsc.addupdate(lacc.at[rid, cs], buf[base + i, cs])

            pl.when(uniform)(fast)
            pl.when(jnp.logical_not(uniform))(slow)
            return carry

        lax.fori_loop(0, N_GROUPS, group_body, 0)

    load_c(0)
    load(1, 0)

    def super_body(g, carry):
        j0 = g * SUPER
        wait_load_c()
        for k in range(N_SUB):
            pltpu.async_copy(
                buf_c.at[pl.ds(k * SUB, SUB)], acc.at[idb_c.at[k]],
                csem, add=True,
            )
        load(j0 + 2, 1)
        wait_load(0)
        process_chunk(bufs[0], idbs[0])
        load(j0 + 3, 0)
        wait_load(1)
        process_chunk(bufs[1], idbs[1])
        load(j0 + 4, 1)
        wait_load(0)
        process_chunk(bufs[0], idbs[0])
        for k in range(N_SUB):
            pltpu.make_async_copy(
                buf_c.at[pl.ds(0, SUB)], acc.at[idb_c.at[0]], csem
            ).wait()
        load_c(jnp.minimum(j0 + SUPER, N_CHUNKS - 1))
        load(jnp.minimum(j0 + SUPER + 1, N_CHUNKS - 1), 0)
        wait_load(1)
        process_chunk(bufs[1], idbs[1])
        return carry

    lax.fori_loop(0, N_SUPER, super_body, 0)
    wait_load_c()
    wait_load(0)

    scatd = [
        pltpu.async_copy(
            lacc.at[pl.ds(r * SCAT, SCAT)], acc.at[sidx.at[r]], ssem,
            add=True,
        )
        for r in range(N_SCAT)
    ]
    for d in scatd:
        d.wait()

    plsc.subcore_barrier()

    pltpu.sync_copy(acc.at[pl.ds(s * ROWS_PER_OUT, ROWS_PER_OUT)], obuf)
    pltpu.sync_copy(
        obuf, out_hbm.at[c, pl.ds(s * ROWS_PER_OUT, ROWS_PER_OUT)]
    )


def kernel(x, batch):
    ids = batch.astype(jnp.int32)
    halves = _seg_sum(
        x, ids.reshape(N // CHUNK, CHUNK), ids.reshape(N // SUB, SUB)
    )
    return jnp.concatenate([halves[0], halves[1]], axis=1)
